# baseline (speedup 1.0000x reference)
import functools

import jax
import jax.numpy as jnp
from jax import lax
from jax.experimental import pallas as pl
from jax.experimental.pallas import tpu as pltpu
from jax.experimental.pallas import tpu_sc as plsc

_V = 462
_E = 462
_D = 256
_L = 16
_EPW = 16
_NC = 2
_NS = 16
_W = 36

_mesh = plsc.VectorSubcoreMesh(core_axis_name="c", subcore_axis_name="s")


@functools.partial(
    pl.kernel,
    mesh=_mesh,
    out_type=jax.ShapeDtypeStruct((2 * _V, _D), jnp.float32),
    scratch_types=[
        pltpu.VMEM((_W,), jnp.int32),
        pltpu.VMEM((2 * _EPW,), jnp.int32),
        pltpu.VMEM((2 * _EPW, _D), jnp.float32),
        pltpu.VMEM((_EPW, _D), jnp.float32),
        pltpu.SemaphoreType.DMA,
        pltpu.SemaphoreType.DMA,
        pltpu.SemaphoreType.DMA,
    ],
)
def _gunpool_sc(x_hbm, pidx_hbm, out_hbm, win_v, idx_v, rows_v,
                copy_v, sem0, sem1, sem2):
    w = lax.axis_index("s") * _NC + lax.axis_index("c")
    base = jnp.minimum(w * _EPW, _E - _EPW)
    lane = lax.broadcasted_iota(jnp.int32, (_L,), 0)
    crow = base + lane

    start = jnp.minimum(w * (2 * _EPW), 2 * _E - _W)
    off = 2 * base - start
    wf = pltpu.async_copy(pidx_hbm.at[pl.ds(start, _W)], win_v, sem1)

    cp_in = pltpu.async_copy(x_hbm.at[crow], copy_v, sem2)

    wf.wait()
    idx_v[pl.ds(0, _L)] = win_v[pl.ds(off, _L)]
    idx_v[pl.ds(_L, _L)] = win_v[pl.ds(off + _L, _L)]
    g = pltpu.async_copy(x_hbm.at[idx_v], rows_v, sem0)

    cp_in.wait()
    cp_out = pltpu.async_copy(copy_v, out_hbm.at[crow], sem2)

    g.wait()
    for i in range(_EPW):
        for j in range(_D // _L):
            s = pl.ds(j * _L, _L)
            rows_v[i, s] = 0.5 * (rows_v[2 * i, s] + rows_v[2 * i + 1, s])

    scat = pltpu.async_copy(rows_v.at[pl.ds(0, _EPW)], out_hbm.at[crow + _V],
                            sem0)
    cp_out.wait()
    scat.wait()


def kernel(input, pool_idx):
    return _gunpool_sc(input, pool_idx.reshape(-1))


# speedup vs baseline: 1.0179x; 1.0179x over previous
import functools

import jax
import jax.numpy as jnp
from jax import lax
from jax.experimental import pallas as pl
from jax.experimental.pallas import tpu as pltpu
from jax.experimental.pallas import tpu_sc as plsc

_V = 462
_E = 462
_D = 256
_L = 16
_EPW = 16
_NC = 2
_NS = 16
_W = 36

_mesh = plsc.VectorSubcoreMesh(core_axis_name="c", subcore_axis_name="s")


@functools.partial(
    pl.kernel,
    mesh=_mesh,
    out_type=jax.ShapeDtypeStruct((2 * _V, _D), jnp.float32),
    scratch_types=[
        pltpu.VMEM((_W,), jnp.int32),
        pltpu.VMEM((2 * _EPW,), jnp.int32),
        pltpu.VMEM((2 * _EPW, _D), jnp.float32),
        pltpu.VMEM((_EPW, _D), jnp.float32),
        pltpu.VMEM((_EPW, _D), jnp.float32),
        pltpu.SemaphoreType.DMA,
        pltpu.SemaphoreType.DMA,
        pltpu.SemaphoreType.DMA,
    ],
)
def _gunpool_sc(x_hbm, pidx_hbm, out_hbm, win_v, idx_v, rows_v, mid_v,
                copy_v, sem0, sem1, sem2):
    w = lax.axis_index("s") * _NC + lax.axis_index("c")
    base = jnp.minimum(w * _EPW, _E - _EPW)
    lane = lax.broadcasted_iota(jnp.int32, (_L,), 0)
    crow = base + lane

    start = jnp.minimum(w * (2 * _EPW), 2 * _E - _W)
    off = 2 * base - start
    wf = pltpu.async_copy(pidx_hbm.at[pl.ds(start, _W)], win_v, sem1)

    cp_in = pltpu.async_copy(x_hbm.at[crow], copy_v, sem2)

    wf.wait()
    idx_v[pl.ds(0, _L)] = win_v[pl.ds(off, _L)]
    idx_v[pl.ds(_L, _L)] = win_v[pl.ds(off + _L, _L)]
    g = pltpu.async_copy(x_hbm.at[idx_v], rows_v, sem0)

    cp_in.wait()
    cp_out = pltpu.async_copy(copy_v, out_hbm.at[crow], sem2)

    g.wait()
    def _avg_row(i, carry):
        for j in range(_D // _L):
            s = pl.ds(j * _L, _L)
            mid_v[i, s] = 0.5 * (rows_v[2 * i, s] + rows_v[2 * i + 1, s])
        return carry

    lax.fori_loop(0, _EPW, _avg_row, 0, unroll=False)

    scat = pltpu.async_copy(mid_v, out_hbm.at[crow + _V], sem0)
    cp_out.wait()
    scat.wait()


def kernel(input, pool_idx):
    return _gunpool_sc(input, pool_idx.reshape(-1))
